# Initial kernel scaffold; baseline (speedup 1.0000x reference)
import jax
import jax.numpy as jnp
from jax.experimental import pallas as pl


def kernel(z, rbf, idx_i, idx_j, node_table, W_rbf, W_edge, b_edge):
    raise NotImplementedError("write your pallas kernel here")


# speedup vs baseline: 2.4111x; 2.4111x over previous
import functools

import jax
import jax.numpy as jnp
from jax import lax
from jax.experimental import pallas as pl
from jax.experimental.pallas import tpu as pltpu
from jax.experimental.pallas import tpu_sc as plsc

N_NODES = 10000
N_EDGES = 320000
D = 128
N_RADIAL = 16
VOCAB_PAD = 128

EDGE_BLOCK = 2560
N_BLOCKS = N_EDGES // EDGE_BLOCK

_UNROLL = 5


def _sc_zgather(z, idx_i, idx_j):
    info = plsc.get_sparse_core_info()
    nc, ns, nl = info.num_cores, info.num_subcores, info.num_lanes
    nw = nc * ns
    per_w = N_EDGES // nw
    steps = per_w // (nl * _UNROLL)
    mesh = plsc.VectorSubcoreMesh(core_axis_name="c", subcore_axis_name="s")

    @functools.partial(
        pl.kernel,
        mesh=mesh,
        out_type=[
            jax.ShapeDtypeStruct((N_EDGES,), jnp.int32),
            jax.ShapeDtypeStruct((N_EDGES,), jnp.int32),
        ],
        scratch_types=[
            pltpu.VMEM((N_NODES,), jnp.int32),
            pltpu.VMEM((per_w,), jnp.int32),
            pltpu.VMEM((per_w,), jnp.int32),
        ],
        compiler_params=pltpu.CompilerParams(needs_layout_passes=False),
    )
    def k(z_hbm, ii_hbm, jj_hbm, zi_hbm, zj_hbm, z_v, idx_v, out_v):
        wid = lax.axis_index("s") * nc + lax.axis_index("c")
        base = wid * per_w
        pltpu.sync_copy(z_hbm, z_v)

        def run(src_hbm, dst_hbm):
            pltpu.sync_copy(src_hbm.at[pl.ds(base, per_w)], idx_v)

            def body(i, _):
                off = i * (nl * _UNROLL)
                for u in range(_UNROLL):
                    iv = idx_v[pl.ds(off + u * nl, nl)]
                    out_v[pl.ds(off + u * nl, nl)] = plsc.load_gather(z_v, [iv])
                return 0

            lax.fori_loop(0, steps, body, 0)
            pltpu.sync_copy(out_v, dst_hbm.at[pl.ds(base, per_w)])

        run(ii_hbm, zi_hbm)
        run(jj_hbm, zj_hbm)

    return k(z, idx_i, idx_j)


def _fold_body(nt_ref, we_ref, wr_ref, a_ref, b_ref, wc_ref):
    nt = nt_ref[...]
    a_ref[...] = jnp.dot(
        nt, we_ref[0:D, :], preferred_element_type=jnp.float32
    ).astype(jnp.bfloat16)
    b_ref[...] = jnp.dot(
        nt, we_ref[D : 2 * D, :], preferred_element_type=jnp.float32
    ).astype(jnp.bfloat16)
    wc_ref[...] = jnp.dot(
        wr_ref[...], we_ref[2 * D : 2 * D + D, :], preferred_element_type=jnp.float32
    )


def _fold_weights(nt_pad, W_edge, W_rbf):
    return pl.pallas_call(
        _fold_body,
        out_shape=[
            jax.ShapeDtypeStruct((VOCAB_PAD, D), jnp.bfloat16),
            jax.ShapeDtypeStruct((VOCAB_PAD, D), jnp.bfloat16),
            jax.ShapeDtypeStruct((N_RADIAL, D), jnp.float32),
        ],
    )(nt_pad, W_edge, W_rbf)


def _mlp_body(zi_ref, zj_ref, rbf_ref, a_ref, b_ref, wc_ref, bias_ref, out_ref):
    cols = lax.broadcasted_iota(jnp.int32, (EDGE_BLOCK, VOCAB_PAD), 1)
    ohj = (zj_ref[...] == cols).astype(jnp.bfloat16)
    ohi = (zi_ref[...] == cols).astype(jnp.bfloat16)
    acc = jnp.dot(ohj, a_ref[...], preferred_element_type=jnp.float32)
    acc += jnp.dot(ohi, b_ref[...], preferred_element_type=jnp.float32)
    acc += jnp.dot(rbf_ref[...], wc_ref[...], preferred_element_type=jnp.float32)
    acc += bias_ref[...]
    out_ref[...] = acc * (1.0 / (1.0 + jnp.exp(-acc)))


def _edge_mlp(zi, zj, rbf, A, B, Wc, bias):
    return pl.pallas_call(
        _mlp_body,
        grid=(N_BLOCKS,),
        in_specs=[
            pl.BlockSpec((EDGE_BLOCK, 1), lambda i: (i, 0)),
            pl.BlockSpec((EDGE_BLOCK, 1), lambda i: (i, 0)),
            pl.BlockSpec((EDGE_BLOCK, N_RADIAL), lambda i: (i, 0)),
            pl.BlockSpec((VOCAB_PAD, D), lambda i: (0, 0)),
            pl.BlockSpec((VOCAB_PAD, D), lambda i: (0, 0)),
            pl.BlockSpec((N_RADIAL, D), lambda i: (0, 0)),
            pl.BlockSpec((1, D), lambda i: (0, 0)),
        ],
        out_specs=pl.BlockSpec((EDGE_BLOCK, D), lambda i: (i, 0)),
        out_shape=jax.ShapeDtypeStruct((N_EDGES, D), jnp.float32),
    )(zi, zj, rbf, A, B, Wc, bias)


def kernel(z, rbf, idx_i, idx_j, node_table, W_rbf, W_edge, b_edge):
    nt_pad = jnp.zeros((VOCAB_PAD, D), jnp.float32).at[: node_table.shape[0]].set(
        node_table
    )
    A, B, Wc = _fold_weights(nt_pad, W_edge, W_rbf)
    zi, zj = _sc_zgather(
        z.astype(jnp.int32), idx_i.astype(jnp.int32), idx_j.astype(jnp.int32)
    )
    out = _edge_mlp(
        zi.reshape(N_EDGES, 1),
        zj.reshape(N_EDGES, 1),
        rbf,
        A,
        B,
        Wc,
        b_edge.reshape(1, D),
    )
    return out


# speedup vs baseline: 5.1878x; 2.1516x over previous
import functools

import jax
import jax.numpy as jnp
from jax import lax
from jax.experimental import pallas as pl
from jax.experimental.pallas import tpu as pltpu
from jax.experimental.pallas import tpu_sc as plsc

N_NODES = 10000
N_EDGES = 320000
D = 128
N_RADIAL = 16
VOCAB_PAD = 128

EDGE_BLOCK = 2560
N_BLOCKS = N_EDGES // EDGE_BLOCK

_UNROLL = 5


def _sc_zgather(z, idx_i, idx_j):
    info = plsc.get_sparse_core_info()
    nc, ns, nl = info.num_cores, info.num_subcores, info.num_lanes
    nw = nc * ns
    per_w = N_EDGES // nw
    steps = per_w // (nl * _UNROLL)
    mesh = plsc.VectorSubcoreMesh(core_axis_name="c", subcore_axis_name="s")

    @functools.partial(
        pl.kernel,
        mesh=mesh,
        out_type=[
            jax.ShapeDtypeStruct((N_EDGES,), jnp.int32),
            jax.ShapeDtypeStruct((N_EDGES,), jnp.int32),
        ],
        scratch_types=[
            pltpu.VMEM((N_NODES,), jnp.int32),
            pltpu.VMEM((per_w,), jnp.int32),
            pltpu.VMEM((per_w,), jnp.int32),
        ],
        compiler_params=pltpu.CompilerParams(needs_layout_passes=False),
    )
    def k(z_hbm, ii_hbm, jj_hbm, zi_hbm, zj_hbm, z_v, idx_v, out_v):
        wid = lax.axis_index("s") * nc + lax.axis_index("c")
        base = wid * per_w
        pltpu.sync_copy(z_hbm, z_v)

        def run(src_hbm, dst_hbm):
            pltpu.sync_copy(src_hbm.at[pl.ds(base, per_w)], idx_v)

            def body(i, _):
                off = i * (nl * _UNROLL)
                for u in range(_UNROLL):
                    iv = idx_v[pl.ds(off + u * nl, nl)]
                    out_v[pl.ds(off + u * nl, nl)] = plsc.load_gather(z_v, [iv])
                return 0

            lax.fori_loop(0, steps, body, 0)
            pltpu.sync_copy(out_v, dst_hbm.at[pl.ds(base, per_w)])

        run(ii_hbm, zi_hbm)
        run(jj_hbm, zj_hbm)

    return k(z, idx_i, idx_j)


def _fold_body(nt_ref, we_ref, wr_ref, a_ref, b_ref, wc_ref):
    nt = nt_ref[...]
    a_ref[...] = jnp.dot(
        nt, we_ref[0:D, :], preferred_element_type=jnp.float32
    ).astype(jnp.bfloat16)
    b_ref[...] = jnp.dot(
        nt, we_ref[D : 2 * D, :], preferred_element_type=jnp.float32
    ).astype(jnp.bfloat16)
    wc_ref[...] = jnp.dot(
        wr_ref[...], we_ref[2 * D : 2 * D + D, :], preferred_element_type=jnp.float32
    )


def _fold_weights(nt_pad, W_edge, W_rbf):
    return pl.pallas_call(
        _fold_body,
        out_shape=[
            jax.ShapeDtypeStruct((VOCAB_PAD, D), jnp.bfloat16),
            jax.ShapeDtypeStruct((VOCAB_PAD, D), jnp.bfloat16),
            jax.ShapeDtypeStruct((N_RADIAL, D), jnp.float32),
        ],
    )(nt_pad, W_edge, W_rbf)


_CHUNKS = EDGE_BLOCK // 128


def _mlp_body(zi_ref, zj_ref, rbf_ref, a_ref, b_ref, wc_ref, bias_ref, out_ref):
    sub = lax.broadcasted_iota(jnp.int32, (VOCAB_PAD, 128), 0)
    zi = zi_ref[0]
    zj = zj_ref[0]
    a = a_ref[...]
    b = b_ref[...]
    dn = (((0,), (0,)), ((), ()))
    acc = jnp.dot(rbf_ref[...], wc_ref[...], preferred_element_type=jnp.float32)
    acc += bias_ref[...]
    parts = []
    for g in range(_CHUNKS):
        ohtj = (zj[g : g + 1, :] == sub).astype(jnp.bfloat16)
        ohti = (zi[g : g + 1, :] == sub).astype(jnp.bfloat16)
        part = lax.dot_general(ohtj, a, dn, preferred_element_type=jnp.float32)
        part += lax.dot_general(ohti, b, dn, preferred_element_type=jnp.float32)
        parts.append(part)
    acc += jnp.concatenate(parts, axis=0)
    out_ref[...] = acc * (1.0 / (1.0 + jnp.exp(-acc)))


def _edge_mlp(zi, zj, rbf, A, B, Wc, bias):
    return pl.pallas_call(
        _mlp_body,
        grid=(N_BLOCKS,),
        in_specs=[
            pl.BlockSpec((1, EDGE_BLOCK // 128, 128), lambda i: (i, 0, 0)),
            pl.BlockSpec((1, EDGE_BLOCK // 128, 128), lambda i: (i, 0, 0)),
            pl.BlockSpec((EDGE_BLOCK, N_RADIAL), lambda i: (i, 0)),
            pl.BlockSpec((VOCAB_PAD, D), lambda i: (0, 0)),
            pl.BlockSpec((VOCAB_PAD, D), lambda i: (0, 0)),
            pl.BlockSpec((N_RADIAL, D), lambda i: (0, 0)),
            pl.BlockSpec((1, D), lambda i: (0, 0)),
        ],
        out_specs=pl.BlockSpec((EDGE_BLOCK, D), lambda i: (i, 0)),
        out_shape=jax.ShapeDtypeStruct((N_EDGES, D), jnp.float32),
    )(zi, zj, rbf, A, B, Wc, bias)


def kernel(z, rbf, idx_i, idx_j, node_table, W_rbf, W_edge, b_edge):
    nt_pad = jnp.zeros((VOCAB_PAD, D), jnp.float32).at[: node_table.shape[0]].set(
        node_table
    )
    A, B, Wc = _fold_weights(nt_pad, W_edge, W_rbf)
    zi, zj = _sc_zgather(
        z.astype(jnp.int32), idx_i.astype(jnp.int32), idx_j.astype(jnp.int32)
    )
    out = _edge_mlp(
        zi.reshape(N_BLOCKS, EDGE_BLOCK // 128, 128),
        zj.reshape(N_BLOCKS, EDGE_BLOCK // 128, 128),
        rbf,
        A,
        B,
        Wc,
        b_edge.reshape(1, D),
    )
    return out
